# speedup vs baseline: 31.4586x; 31.4586x over previous
import jax
import jax.numpy as jnp
from jax import lax
from jax.experimental import pallas as pl

_H = 256
_W = 256
_K = 18
_BH = 64


def _box_kernel(rlo_ref, clo_ref, out_ref):
    wk = _W * _K
    lane = lax.broadcasted_iota(jnp.int32, (1, wk), 1)
    j_id = lane // _K
    cd = (j_id - clo_ref[0]).astype(jnp.uint32)
    colf = jnp.where(cd <= 10, jnp.float32(1.0), jnp.float32(0.0))
    base = pl.program_id(1) * _BH
    ri = base + lax.broadcasted_iota(jnp.int32, (_BH, wk), 0)
    rd = (ri - rlo_ref[0]).astype(jnp.uint32)
    out_ref[0] = jnp.where(rd <= 10, colf, jnp.float32(0.0))


def kernel(x):
    b, k, _ = x.shape
    wk = _W * _K
    rlo = jnp.broadcast_to((x[:, :, 0] - 5)[:, None, :], (b, _W, k))
    clo = jnp.broadcast_to((x[:, :, 1] - 5)[:, None, :], (b, _W, k))
    rlo = rlo.reshape(b, 1, wk)
    clo = clo.reshape(b, 1, wk)

    out = pl.pallas_call(
        _box_kernel,
        grid=(b, _H // _BH),
        in_specs=[
            pl.BlockSpec((1, 1, wk), lambda bi, hi: (bi, 0, 0)),
            pl.BlockSpec((1, 1, wk), lambda bi, hi: (bi, 0, 0)),
        ],
        out_specs=pl.BlockSpec((1, _BH, wk), lambda bi, hi: (bi, hi, 0)),
        out_shape=jax.ShapeDtypeStruct((b, _H, wk), jnp.float32),
    )(rlo, clo)
    return out.reshape(b, _H, _W, k)
